# baseline (speedup 1.0000x reference)
import jax
import jax.numpy as jnp
from jax import lax
from jax.experimental import pallas as pl
from jax.experimental.pallas import tpu as pltpu
from jax.experimental.pallas import tpu_sc as plsc
from jax._src.pallas import mpmd

SEQ_LEN = 2048
D_MODEL = 1024
BATCH = 32

NUM_CORES = 2
NUM_SUBCORES = 16
NUM_WORKERS = NUM_CORES * NUM_SUBCORES

K_SCS = 128
T_TEC = SEQ_LEN - NUM_CORES * K_SCS
ROWS_PER_TEC = T_TEC // NUM_WORKERS
TEC_CHUNK = ROWS_PER_TEC

SCS_CHUNK = 16
REP_S = 4
SCS_NCHUNK = K_SCS // SCS_CHUNK

_vec_mesh = plsc.VectorSubcoreMesh(
    core_axis_name="c", subcore_axis_name="s",
    num_cores=NUM_CORES, num_subcores=NUM_SUBCORES,
)
_scs_mesh = plsc.ScalarSubcoreMesh(axis_name="c", num_cores=NUM_CORES)


def _tec_fn(pe_hbm, out_hbm, tbuf0, tbuf1, tsem0, tsem1, sbuf0, sbuf1, ssem):
    del sbuf0, sbuf1, ssem
    wid = lax.axis_index("s") * NUM_CORES + lax.axis_index("c")
    base = wid * ROWS_PER_TEC

    def fill(buf, row0, sem):
        pltpu.sync_copy(pe_hbm.at[pl.ds(row0, TEC_CHUNK)], buf.at[:, 0])
        for n in range(BATCH):
            pltpu.async_copy(
                buf, out_hbm.at[pl.ds(row0, TEC_CHUNK), pl.ds(n, 1)], sem)

    def drain(buf, row0, sem):
        for n in range(BATCH):
            pltpu.make_async_copy(
                buf, out_hbm.at[pl.ds(row0, TEC_CHUNK), pl.ds(n, 1)], sem).wait()

    del tbuf1, tsem1
    fill(tbuf0, base, tsem0)
    drain(tbuf0, base, tsem0)


def _scs_fn(pe_hbm, out_hbm, tbuf0, tbuf1, tsem0, tsem1, sbuf0, sbuf1, ssem):
    del tbuf0, tbuf1, tsem0, tsem1
    cid = lax.axis_index("c")
    base = T_TEC + cid * K_SCS

    def fill(buf, row0, sem):
        for r in range(REP_S):
            pltpu.sync_copy(pe_hbm.at[pl.ds(row0, SCS_CHUNK)], buf.at[:, r])
        for n in range(BATCH // REP_S):
            pltpu.async_copy(
                buf, out_hbm.at[pl.ds(row0, SCS_CHUNK), pl.ds(REP_S * n, REP_S)],
                sem)

    def drain(buf, row0, sem):
        for n in range(BATCH // REP_S):
            pltpu.make_async_copy(
                buf, out_hbm.at[pl.ds(row0, SCS_CHUNK), pl.ds(REP_S * n, REP_S)],
                sem).wait()

    bufs = (sbuf0, sbuf1)
    for c in range(SCS_NCHUNK):
        if c >= 2:
            drain(bufs[c % 2], base + (c - 2) * SCS_CHUNK, ssem)
        fill(bufs[c % 2], base + c * SCS_CHUNK, ssem)
    for c in range(max(SCS_NCHUNK - 2, 0), SCS_NCHUNK):
        drain(bufs[c % 2], base + c * SCS_CHUNK, ssem)


_hybrid = mpmd.mpmd_map(
    [(_scs_mesh, _scs_fn), (_vec_mesh, _tec_fn)],
    out_types=jax.ShapeDtypeStruct((SEQ_LEN, BATCH, D_MODEL), jnp.float32),
    scratch_types=[
        (pltpu.VMEM @ _vec_mesh)((TEC_CHUNK, 1, D_MODEL), jnp.float32),
        (pltpu.VMEM @ _vec_mesh)((TEC_CHUNK, 1, D_MODEL), jnp.float32),
        pltpu.SemaphoreType.DMA @ _vec_mesh,
        pltpu.SemaphoreType.DMA @ _vec_mesh,
        pltpu.VMEM_SHARED((SCS_CHUNK, REP_S, D_MODEL), jnp.float32),
        pltpu.VMEM_SHARED((SCS_CHUNK, REP_S, D_MODEL), jnp.float32),
        pltpu.SemaphoreType.DMA @ _scs_mesh,
    ],
)


def kernel(z, pos_embed):
    del z
    return _hybrid(pos_embed)


# speedup vs baseline: 1.0208x; 1.0208x over previous
import functools

import jax
import jax.numpy as jnp
from jax import lax
from jax.experimental import pallas as pl
from jax.experimental.pallas import tpu as pltpu
from jax.experimental.pallas import tpu_sc as plsc

SEQ_LEN = 2048
D_MODEL = 1024
BATCH = 32

NUM_CORES = 2
NUM_SUBCORES = 16
NUM_WORKERS = NUM_CORES * NUM_SUBCORES
ROWS_PER_W = SEQ_LEN // NUM_WORKERS

_mesh = plsc.VectorSubcoreMesh(
    core_axis_name="c", subcore_axis_name="s",
    num_cores=NUM_CORES, num_subcores=NUM_SUBCORES,
)


@functools.partial(
    pl.kernel,
    out_type=jax.ShapeDtypeStruct((SEQ_LEN, BATCH, D_MODEL), jnp.float32),
    mesh=_mesh,
    scratch_types=[
        pltpu.VMEM((ROWS_PER_W, D_MODEL), jnp.float32),
        pltpu.SemaphoreType.DMA,
    ],
)
def _sc_broadcast(pe_hbm, out_hbm, buf, sem):
    wid = lax.axis_index("s") * NUM_CORES + lax.axis_index("c")
    base = wid * ROWS_PER_W
    pltpu.sync_copy(pe_hbm.at[pl.ds(base, ROWS_PER_W)], buf)
    for n in range(BATCH):
        pltpu.async_copy(buf, out_hbm.at[pl.ds(base, ROWS_PER_W), n], sem)
    for n in range(BATCH):
        pltpu.make_async_copy(buf, out_hbm.at[pl.ds(base, ROWS_PER_W), n], sem).wait()


def kernel(z, pos_embed):
    del z
    return _sc_broadcast(pos_embed)


# speedup vs baseline: 1.0231x; 1.0023x over previous
import functools

import jax
import jax.numpy as jnp
from jax import lax
from jax.experimental import pallas as pl
from jax.experimental.pallas import tpu as pltpu
from jax.experimental.pallas import tpu_sc as plsc

SEQ_LEN = 2048
D_MODEL = 1024
BATCH = 32

NUM_CORES = 2
NUM_SUBCORES = 16
NUM_WORKERS = NUM_CORES * NUM_SUBCORES
ROWS_PER_W = SEQ_LEN // NUM_WORKERS

_mesh = plsc.VectorSubcoreMesh(
    core_axis_name="c", subcore_axis_name="s",
    num_cores=NUM_CORES, num_subcores=NUM_SUBCORES,
)


@functools.partial(
    pl.kernel,
    out_type=jax.ShapeDtypeStruct((SEQ_LEN, BATCH, D_MODEL), jnp.float32),
    mesh=_mesh,
    scratch_types=[
        pltpu.VMEM((ROWS_PER_W, D_MODEL), jnp.float32),
        pltpu.SemaphoreType.DMA,
    ],
)
def _sc_broadcast(pe_hbm, out_hbm, buf, sem):
    wid = lax.axis_index("c") * NUM_SUBCORES + lax.axis_index("s")
    base = wid * ROWS_PER_W
    pltpu.sync_copy(pe_hbm.at[pl.ds(base, ROWS_PER_W)], buf)
    for n in range(BATCH):
        pltpu.async_copy(buf, out_hbm.at[pl.ds(base, ROWS_PER_W), n], sem)
    for n in range(BATCH):
        pltpu.make_async_copy(buf, out_hbm.at[pl.ds(base, ROWS_PER_W), n], sem).wait()


def kernel(z, pos_embed):
    del z
    return _sc_broadcast(pos_embed)
